# Initial kernel scaffold; baseline (speedup 1.0000x reference)
import functools

import jax
import jax.numpy as jnp
import numpy as np
from jax import lax
from jax.experimental import pallas as pl
from jax.experimental.pallas import tpu as pltpu
from jax.experimental.pallas import tpu_sc as plsc

_M = 4096
_IDX_CHUNK = 128


@functools.lru_cache(maxsize=None)
def _consts(B, L, NW):
    P = (B * _M) // NW
    with jax.ensure_compile_time_eval():
        key = jax.random.key(42)
        gidx, nsel = [], []
        for i in range(B):
            kn = jax.random.fold_in(key, 2 * i)
            ki = jax.random.fold_in(key, 2 * i + 1)
            noise = jax.random.normal(kn, (L, 3), jnp.float32)
            idx = jax.random.randint(ki, (_M,), 0, L)
            gidx.append(idx + i * L)
            nsel.append(noise[idx])
        gidx = jnp.concatenate(gidx).astype(jnp.int32)
        gidx = gidx.reshape(NW, P // _IDX_CHUNK, _IDX_CHUNK)
        nsel = jnp.concatenate(nsel, axis=0).reshape(NW, P, 3)
        nsel = nsel.transpose(0, 2, 1)
        return np.asarray(gidx), np.asarray(nsel)


@functools.lru_cache(maxsize=None)
def _make_sc_fn(B, L, NC, NS):
    NW = NC * NS
    P = (B * _M) // NW
    K = P // _IDX_CHUNK
    mesh = plsc.VectorSubcoreMesh(core_axis_name="c", subcore_axis_name="s")

    @functools.partial(
        pl.kernel,
        out_type=jax.ShapeDtypeStruct((B * _M, 3), jnp.float32),
        mesh=mesh,
        scratch_types=[
            pltpu.VMEM((K, _IDX_CHUNK), jnp.int32),
            pltpu.VMEM((P, 3), jnp.float32),
            pltpu.VMEM((P,), jnp.float32),
            pltpu.VMEM((3, P), jnp.float32),
            pltpu.VMEM((P, 3), jnp.float32),
            pltpu.SemaphoreType.DMA,
        ],
    )
    def sc_fn(rows_hbm, r_hbm, gidx_hbm, noise_hbm, out_hbm,
              idx_v, rows_v, r_v, noise_v, out_v, sem):
        wid = lax.axis_index("s") * NC + lax.axis_index("c")
        pltpu.sync_copy(gidx_hbm.at[wid], idx_v)
        copies = []
        for k in range(K):
            copies.append(pltpu.async_copy(
                rows_hbm.at[idx_v.at[k]],
                rows_v.at[pl.ds(k * _IDX_CHUNK, _IDX_CHUNK)], sem))
            copies.append(pltpu.async_copy(
                r_hbm.at[idx_v.at[k]],
                r_v.at[pl.ds(k * _IDX_CHUNK, _IDX_CHUNK)], sem))
        pltpu.sync_copy(noise_hbm.at[wid], noise_v)
        for cpy in copies:
            cpy.wait()

        def body(k, carry):
            jj = lax.iota(jnp.int32, 16) + k * 16
            s16 = jnp.sqrt(r_v[pl.ds(k * 16, 16)])
            for c in range(3):
                cc = jnp.full((16,), c, jnp.int32)
                p16 = plsc.load_gather(rows_v, [jj, cc])
                n16 = noise_v[c, pl.ds(k * 16, 16)]
                plsc.store_scatter(out_v, [jj, cc], s16 * n16 + p16)
            return carry

        lax.fori_loop(0, P // 16, body, 0)
        pltpu.sync_copy(out_v, out_hbm.at[pl.ds(wid * P, P)])

    return sc_fn


def kernel(coords, radius, maxpoints):
    B, L, _ = coords.shape
    info = plsc.get_sparse_core_info()
    NC, NS = info.num_cores, info.num_subcores
    gidx, nsel = _consts(B, L, NC * NS)
    sc_fn = _make_sc_fn(B, L, NC, NS)
    out = sc_fn(coords.reshape(B * L, 3), radius.reshape(B * L),
                jnp.asarray(gidx), jnp.asarray(nsel))
    return out.reshape(B, _M, 3)


# speedup vs baseline: 3.8686x; 3.8686x over previous
import functools

import jax
import jax.numpy as jnp
import numpy as np
from jax import lax
from jax.experimental import pallas as pl
from jax.experimental.pallas import tpu as pltpu
from jax.experimental.pallas import tpu_sc as plsc

_M = 4096
_IDX_CHUNK = 128


def _threefry2x32(k1, k2, x0, x1):
    rot0, rot1 = (13, 15, 26, 6), (17, 29, 16, 24)
    ks0, ks1 = np.uint32(k1), np.uint32(k2)
    ks2 = np.uint32(ks0 ^ ks1 ^ np.uint32(0x1BD11BDA))

    def rounds(x0, x1, rots):
        for r in rots:
            x0 = (x0 + x1).astype(np.uint32)
            x1 = ((x1 << np.uint32(r)) | (x1 >> np.uint32(32 - r))).astype(np.uint32)
            x1 = (x1 ^ x0).astype(np.uint32)
        return x0, x1

    x0 = (np.asarray(x0, np.uint32) + ks0).astype(np.uint32)
    x1 = (np.asarray(x1, np.uint32) + ks1).astype(np.uint32)
    for i, (rots, ka, kb) in enumerate(
            [(rot0, ks1, ks2), (rot1, ks2, ks0), (rot0, ks0, ks1),
             (rot1, ks1, ks2), (rot0, ks2, ks0)]):
        x0, x1 = rounds(x0, x1, rots)
        x0 = (x0 + ka).astype(np.uint32)
        x1 = (x1 + kb + np.uint32(i + 1)).astype(np.uint32)
    return x0, x1


def _fold_in(key, data):
    o0, o1 = _threefry2x32(key[0], key[1], np.uint32(0), np.uint32(data))
    return np.uint32(o0), np.uint32(o1)


def _random_bits(key, n):
    lo = np.arange(n, dtype=np.uint32)
    hi = np.zeros(n, dtype=np.uint32)
    b1, b2 = _threefry2x32(key[0], key[1], hi, lo)
    return (b1 ^ b2).astype(np.uint32)


def _erfinv(x):
    x = np.asarray(x, np.float64)
    w = -np.log((1.0 - x) * (1.0 + x))
    ws, wb = w - 2.5, np.sqrt(np.maximum(w, 5.0)) - 3.0
    cs = [2.81022636e-08, 3.43273939e-07, -3.5233877e-06, -4.39150654e-06,
          0.00021858087, -0.00125372503, -0.00417768164, 0.246640727, 1.50140941]
    cb = [-0.000200214257, 0.000100950558, 0.00134934322, -0.00367342844,
          0.00573950773, -0.0076224613, 0.00943887047, 1.00167406, 2.83297682]
    ps = np.zeros_like(x)
    pb = np.zeros_like(x)
    for c in cs:
        ps = ps * ws + c
    for c in cb:
        pb = pb * wb + c
    return np.where(w < 5.0, ps, pb) * x


def _normal(key, n):
    bits = _random_bits(key, n)
    fb = ((bits >> np.uint32(9)) | np.uint32(0x3F800000)).astype(np.uint32)
    floats = fb.view(np.float32) - np.float32(1.0)
    lo = np.nextafter(np.float32(-1.0), np.float32(0.0), dtype=np.float32)
    hi = np.float32(1.0)
    u = np.maximum(lo, (floats * (hi - lo) + lo).astype(np.float32))
    return (np.sqrt(2.0) * _erfinv(u)).astype(np.float32)


def _randint(key, n, span_):
    hi = np.zeros(2, dtype=np.uint32)
    lo2 = np.arange(2, dtype=np.uint32)
    b1, b2 = _threefry2x32(key[0], key[1], hi, lo2)
    ka, kb = (b1[0], b2[0]), (b1[1], b2[1])
    higher, lower = _random_bits(ka, n), _random_bits(kb, n)
    span = np.uint32(span_)
    mult = np.uint32(np.uint32(2 ** 16) % span)
    mult = np.uint32((np.uint64(mult) * np.uint64(mult)) % span)
    off = ((higher % span) * mult + (lower % span)).astype(np.uint32)
    return (off % span).astype(np.int32)


@functools.lru_cache(maxsize=None)
def _consts(B, L, NW):
    P = (B * _M) // NW
    K = P // _IDX_CHUNK
    key = (np.uint32(0), np.uint32(42))
    gidx, nsel = [], []
    with np.errstate(over="ignore"):
        for i in range(B):
            kn = _fold_in(key, 2 * i)
            ki = _fold_in(key, 2 * i + 1)
            noise = _normal(kn, L * 3).reshape(L, 3)
            idx = _randint(ki, _M, L)
            gidx.append(idx.astype(np.int64) + i * L)
            nsel.append(noise[idx])
    gidx = np.concatenate(gidx).reshape(NW, P)
    idx3 = (3 * gidx[:, None, :] + np.arange(3)[None, :, None])
    idx3 = idx3.reshape(NW, 3 * K, _IDX_CHUNK).astype(np.int32)
    idxr = gidx.reshape(NW, K, _IDX_CHUNK).astype(np.int32)
    nsel = np.concatenate(nsel, axis=0).reshape(NW, P, 3)
    nsel = np.ascontiguousarray(nsel.transpose(0, 2, 1)).reshape(NW, 3 * P)
    return idx3, idxr, nsel


@functools.lru_cache(maxsize=None)
def _make_sc_fn(B, L, NC, NS):
    NW = NC * NS
    P = (B * _M) // NW
    K = P // _IDX_CHUNK
    mesh = plsc.VectorSubcoreMesh(core_axis_name="c", subcore_axis_name="s")

    @functools.partial(
        pl.kernel,
        out_type=jax.ShapeDtypeStruct((3 * B * _M,), jnp.float32),
        mesh=mesh,
        scratch_types=[
            pltpu.VMEM((3 * K, _IDX_CHUNK), jnp.int32),
            pltpu.VMEM((K, _IDX_CHUNK), jnp.int32),
            pltpu.VMEM((3 * P,), jnp.float32),
            pltpu.VMEM((P,), jnp.float32),
            pltpu.VMEM((3 * P,), jnp.float32),
            pltpu.VMEM((3 * P,), jnp.float32),
            pltpu.SemaphoreType.DMA,
        ],
    )
    def sc_fn(cf_hbm, r_hbm, idx3_hbm, idxr_hbm, noise_hbm, out_hbm,
              idx3_v, idxr_v, p_v, r_v, noise_v, out_v, sem):
        wid = lax.axis_index("s") * NC + lax.axis_index("c")
        pltpu.sync_copy(idx3_hbm.at[wid], idx3_v)
        pltpu.sync_copy(idxr_hbm.at[wid], idxr_v)
        copies = []
        for k in range(3 * K):
            copies.append(pltpu.async_copy(
                cf_hbm.at[idx3_v.at[k]],
                p_v.at[pl.ds(k * _IDX_CHUNK, _IDX_CHUNK)], sem))
        for k in range(K):
            copies.append(pltpu.async_copy(
                r_hbm.at[idxr_v.at[k]],
                r_v.at[pl.ds(k * _IDX_CHUNK, _IDX_CHUNK)], sem))
        pltpu.sync_copy(noise_hbm.at[wid], noise_v)
        for cpy in copies:
            cpy.wait()

        def sc_sqrt(x):
            i = lax.bitcast_convert_type(x, jnp.int32)
            y = lax.bitcast_convert_type((i >> 1) + jnp.int32(0x1FBD1DF5),
                                         jnp.float32)
            for _ in range(3):
                y = jnp.float32(0.5) * (y + x / y)
            return y

        def body(k, carry):
            s16 = sc_sqrt(r_v[pl.ds(k * 16, 16)])
            for c in range(3):
                n16 = noise_v[pl.ds(c * P + k * 16, 16)]
                p16 = p_v[pl.ds(c * P + k * 16, 16)]
                out_v[pl.ds(c * P + k * 16, 16)] = s16 * n16 + p16
            return carry

        lax.fori_loop(0, P // 16, body, 0)
        for c in range(3):
            pltpu.sync_copy(out_v.at[pl.ds(c * P, P)],
                            out_hbm.at[pl.ds(c * B * _M + wid * P, P)])

    return sc_fn


def kernel(coords, radius, maxpoints):
    B, L, _ = coords.shape
    info = plsc.get_sparse_core_info()
    NC, NS = info.num_cores, info.num_subcores
    idx3, idxr, nsel = _consts(B, L, NC * NS)
    sc_fn = _make_sc_fn(B, L, NC, NS)
    flat = sc_fn(coords.reshape(B * L * 3), radius.reshape(B * L),
                 jnp.asarray(idx3), jnp.asarray(idxr), jnp.asarray(nsel))
    return flat.reshape(3, B * _M).T.reshape(B, _M, 3)


# speedup vs baseline: 14.3587x; 3.7117x over previous
import functools

import jax
import jax.numpy as jnp
import numpy as np
from jax import lax
from jax.experimental import pallas as pl
from jax.experimental.pallas import tpu as pltpu
from jax.experimental.pallas import tpu_sc as plsc

_M = 4096
_IDX_CHUNK = 128


def _threefry2x32(k1, k2, x0, x1):
    rot0, rot1 = (13, 15, 26, 6), (17, 29, 16, 24)
    ks0, ks1 = np.uint32(k1), np.uint32(k2)
    ks2 = np.uint32(ks0 ^ ks1 ^ np.uint32(0x1BD11BDA))

    def rounds(x0, x1, rots):
        for r in rots:
            x0 = (x0 + x1).astype(np.uint32)
            x1 = ((x1 << np.uint32(r)) | (x1 >> np.uint32(32 - r))).astype(np.uint32)
            x1 = (x1 ^ x0).astype(np.uint32)
        return x0, x1

    x0 = (np.asarray(x0, np.uint32) + ks0).astype(np.uint32)
    x1 = (np.asarray(x1, np.uint32) + ks1).astype(np.uint32)
    for i, (rots, ka, kb) in enumerate(
            [(rot0, ks1, ks2), (rot1, ks2, ks0), (rot0, ks0, ks1),
             (rot1, ks1, ks2), (rot0, ks2, ks0)]):
        x0, x1 = rounds(x0, x1, rots)
        x0 = (x0 + ka).astype(np.uint32)
        x1 = (x1 + kb + np.uint32(i + 1)).astype(np.uint32)
    return x0, x1


def _fold_in(key, data):
    o0, o1 = _threefry2x32(key[0], key[1], np.uint32(0), np.uint32(data))
    return np.uint32(o0), np.uint32(o1)


def _random_bits(key, n):
    lo = np.arange(n, dtype=np.uint32)
    hi = np.zeros(n, dtype=np.uint32)
    b1, b2 = _threefry2x32(key[0], key[1], hi, lo)
    return (b1 ^ b2).astype(np.uint32)


def _erfinv(x):
    x = np.asarray(x, np.float64)
    w = -np.log((1.0 - x) * (1.0 + x))
    ws, wb = w - 2.5, np.sqrt(np.maximum(w, 5.0)) - 3.0
    cs = [2.81022636e-08, 3.43273939e-07, -3.5233877e-06, -4.39150654e-06,
          0.00021858087, -0.00125372503, -0.00417768164, 0.246640727, 1.50140941]
    cb = [-0.000200214257, 0.000100950558, 0.00134934322, -0.00367342844,
          0.00573950773, -0.0076224613, 0.00943887047, 1.00167406, 2.83297682]
    ps = np.zeros_like(x)
    pb = np.zeros_like(x)
    for c in cs:
        ps = ps * ws + c
    for c in cb:
        pb = pb * wb + c
    return np.where(w < 5.0, ps, pb) * x


def _normal(key, n):
    bits = _random_bits(key, n)
    fb = ((bits >> np.uint32(9)) | np.uint32(0x3F800000)).astype(np.uint32)
    floats = fb.view(np.float32) - np.float32(1.0)
    lo = np.nextafter(np.float32(-1.0), np.float32(0.0), dtype=np.float32)
    hi = np.float32(1.0)
    u = np.maximum(lo, (floats * (hi - lo) + lo).astype(np.float32))
    return (np.sqrt(2.0) * _erfinv(u)).astype(np.float32)


def _randint(key, n, span_):
    hi = np.zeros(2, dtype=np.uint32)
    lo2 = np.arange(2, dtype=np.uint32)
    b1, b2 = _threefry2x32(key[0], key[1], hi, lo2)
    ka, kb = (b1[0], b2[0]), (b1[1], b2[1])
    higher, lower = _random_bits(ka, n), _random_bits(kb, n)
    span = np.uint32(span_)
    mult = np.uint32(np.uint32(2 ** 16) % span)
    mult = np.uint32((np.uint64(mult) * np.uint64(mult)) % span)
    off = ((higher % span) * mult + (lower % span)).astype(np.uint32)
    return (off % span).astype(np.int32)


@functools.lru_cache(maxsize=None)
def _consts(B, L, NW):
    P = (B * _M) // NW
    K = P // _IDX_CHUNK
    key = (np.uint32(0), np.uint32(42))
    gidx, nsel = [], []
    with np.errstate(over="ignore"):
        for i in range(B):
            kn = _fold_in(key, 2 * i)
            ki = _fold_in(key, 2 * i + 1)
            noise = _normal(kn, L * 3).reshape(L, 3)
            idx = _randint(ki, _M, L)
            gidx.append(idx.astype(np.int64) + i * L)
            nsel.append(noise[idx])
    gidx = np.concatenate(gidx).reshape(NW, P)
    idx3 = (B * L * np.arange(3)[None, :, None] + gidx[:, None, :])
    idx3 = idx3.reshape(NW, 3 * K, _IDX_CHUNK).astype(np.int32)
    idxr = gidx.reshape(NW, K, _IDX_CHUNK).astype(np.int32)
    nsel = np.concatenate(nsel, axis=0).reshape(NW, P, 3)
    nsel = np.ascontiguousarray(nsel.transpose(0, 2, 1)).reshape(NW, 3 * P)
    return idx3, idxr, nsel


@functools.lru_cache(maxsize=None)
def _make_sc_fn(B, L, NC, NS):
    NW = NC * NS
    P = (B * _M) // NW
    K = P // _IDX_CHUNK
    mesh = plsc.VectorSubcoreMesh(core_axis_name="c", subcore_axis_name="s")

    @functools.partial(
        pl.kernel,
        out_type=jax.ShapeDtypeStruct((3 * B * _M,), jnp.float32),
        mesh=mesh,
        scratch_types=[
            pltpu.VMEM((3 * K, _IDX_CHUNK), jnp.int32),
            pltpu.VMEM((K, _IDX_CHUNK), jnp.int32),
            pltpu.VMEM((3 * P,), jnp.float32),
            pltpu.VMEM((P,), jnp.float32),
            pltpu.VMEM((3 * P,), jnp.float32),
            pltpu.VMEM((3 * P,), jnp.float32),
            pltpu.SemaphoreType.DMA,
        ],
    )
    def sc_fn(cf_hbm, r_hbm, idx3_hbm, idxr_hbm, noise_hbm, out_hbm,
              idx3_v, idxr_v, p_v, r_v, noise_v, out_v, sem):
        wid = lax.axis_index("s") * NC + lax.axis_index("c")
        pltpu.sync_copy(idx3_hbm.at[wid], idx3_v)
        pltpu.sync_copy(idxr_hbm.at[wid], idxr_v)
        copies = []
        for k in range(3 * K):
            copies.append(pltpu.async_copy(
                cf_hbm.at[idx3_v.at[k]],
                p_v.at[pl.ds(k * _IDX_CHUNK, _IDX_CHUNK)], sem))
        for k in range(K):
            copies.append(pltpu.async_copy(
                r_hbm.at[idxr_v.at[k]],
                r_v.at[pl.ds(k * _IDX_CHUNK, _IDX_CHUNK)], sem))
        pltpu.sync_copy(noise_hbm.at[wid], noise_v)
        for cpy in copies:
            cpy.wait()

        def sc_sqrt(x):
            i = lax.bitcast_convert_type(x, jnp.int32)
            y = lax.bitcast_convert_type((i >> 1) + jnp.int32(0x1FBD1DF5),
                                         jnp.float32)
            for _ in range(3):
                y = jnp.float32(0.5) * (y + x / y)
            return y

        def body(k, carry):
            s16 = sc_sqrt(r_v[pl.ds(k * 16, 16)])
            for c in range(3):
                n16 = noise_v[pl.ds(c * P + k * 16, 16)]
                p16 = p_v[pl.ds(c * P + k * 16, 16)]
                out_v[pl.ds(c * P + k * 16, 16)] = s16 * n16 + p16
            return carry

        lax.fori_loop(0, P // 16, body, 0)
        for c in range(3):
            pltpu.sync_copy(out_v.at[pl.ds(c * P, P)],
                            out_hbm.at[pl.ds(c * B * _M + wid * P, P)])

    return sc_fn


def kernel(coords, radius, maxpoints):
    B, L, _ = coords.shape
    info = plsc.get_sparse_core_info()
    NC, NS = info.num_cores, info.num_subcores
    idx3, idxr, nsel = _consts(B, L, NC * NS)
    sc_fn = _make_sc_fn(B, L, NC, NS)
    flat = sc_fn(coords.transpose(2, 0, 1).reshape(3 * B * L),
                 radius.reshape(B * L),
                 jnp.asarray(idx3), jnp.asarray(idxr), jnp.asarray(nsel))
    return flat.reshape(3, B * _M).T.reshape(B, _M, 3)
